# baseline (speedup 1.0000x reference)
import functools

import jax
import jax.numpy as jnp
from jax import lax
from jax.experimental import pallas as pl
from jax.experimental.pallas import tpu as pltpu
from jax.experimental.pallas import tpu_sc as plsc

_N = 10000
_E = 320000
_D = 128
_NPAD = 10240
_DP = 144
_CH = 64
_NCHUNK = _E // _CH
_NC, _NS = 2, 16
_NW = _NC * _NS
_G = 40
_NG = 4
_KMAX = _G * _NG
_CPAD = _NW * _KMAX
_NBUF = 3
_RB = 256


def _mesh():
    return plsc.VectorSubcoreMesh(
        core_axis_name="c", subcore_axis_name="s",
        num_cores=_NC, num_subcores=_NS)


def _sc_deg(row, z, oh):
    @functools.partial(
        pl.kernel,
        out_type=jax.ShapeDtypeStruct((_NC, _NPAD, 16), jnp.float32),
        mesh=_mesh(),
        compiler_params=pltpu.CompilerParams(use_tc_tiling_on_sc=False),
        scratch_types=[
            pltpu.VMEM((_KMAX, _CH), jnp.int32),
            pltpu.VMEM((_CH, 16), jnp.float32),
            pltpu.VMEM_SHARED((_NPAD, 16), jnp.float32),
            pltpu.SemaphoreType.DMA,
        ],
    )
    def k(row_hbm, z_hbm, oh_hbm, out_hbm, idxrall, stage, dacc, sem):
        cid = lax.axis_index("c")
        sid = lax.axis_index("s")
        wid = cid * _NS + sid
        rpt = _NPAD // _NS
        base = wid * _KMAX
        nk = jnp.minimum(_KMAX, _NCHUNK - base)
        pltpu.sync_copy(row_hbm.at[pl.ds(base, _KMAX)], idxrall)
        pltpu.sync_copy(z_hbm, stage)
        for j in range(rpt // _CH):
            pltpu.sync_copy(stage, dacc.at[pl.ds(sid * rpt + j * _CH, _CH)])
        pltpu.sync_copy(oh_hbm, stage)
        plsc.subcore_barrier()

        def body(kk, carry):
            @pl.when(kk < nk)
            def _():
                pltpu.async_copy(stage, dacc.at[idxrall.at[kk]], sem, add=True)

            return carry

        lax.fori_loop(0, _KMAX, body, 0)

        def drain(kk, carry):
            @pl.when(kk < nk)
            def _():
                pltpu.make_async_copy(stage, dacc.at[idxrall.at[0]],
                                      sem).wait()

            return carry

        lax.fori_loop(0, _KMAX, drain, 0)
        plsc.subcore_barrier()
        for j in range(rpt // _CH):
            base = sid * rpt + j * _CH
            pltpu.sync_copy(dacc.at[pl.ds(base, _CH)], stage)
            pltpu.sync_copy(stage, out_hbm.at[cid, pl.ds(base, _CH)])

    return k(row, z, oh)


_NBB = 2 * _NBUF


def _sc_spmm(ypa, ypb, row, col, z):
    @functools.partial(
        pl.kernel,
        out_type=[
            jax.ShapeDtypeStruct((_NC, _NPAD, _D), jnp.float32),
            jax.ShapeDtypeStruct((_NC, _NPAD, 16), jnp.float32),
        ],
        mesh=_mesh(),
        compiler_params=pltpu.CompilerParams(use_tc_tiling_on_sc=False),
        scratch_types=[
            pltpu.VMEM((_G, _CH), jnp.int32),
            pltpu.VMEM((_G, _CH), jnp.int32),
            pltpu.VMEM((_NBUF, _CH, _D), jnp.float32),
            pltpu.VMEM((_NBB, _CH, 16), jnp.float32),
            pltpu.VMEM_SHARED((_NPAD, _D), jnp.float32),
            pltpu.VMEM_SHARED((_NPAD, 16), jnp.float32),
        ] + [pltpu.SemaphoreType.DMA] * (_NBUF + _NBB),
    )
    def k(ypa_hbm, ypb_hbm, row_hbm, col_hbm, z_hbm, outa_hbm, outb_hbm,
          idxr_g, idxc_g, sta, stb, acca, accb, *sems):
        gsems = sems[:_NBUF]
        bsems = sems[_NBUF:]
        cid = lax.axis_index("c")
        sid = lax.axis_index("s")
        wid = cid * _NS + sid
        rpt = _NPAD // _NS
        base = wid * _KMAX
        nk = jnp.minimum(_KMAX, _NCHUNK - base)
        pltpu.sync_copy(z_hbm, sta.at[0])
        pltpu.sync_copy(z_hbm.at[:, pl.ds(0, 16)], stb.at[0])
        for j in range(rpt // _CH):
            rb = sid * rpt + j * _CH
            pltpu.sync_copy(sta.at[0], acca.at[pl.ds(rb, _CH)])
            pltpu.sync_copy(stb.at[0], accb.at[pl.ds(rb, _CH)])
        plsc.subcore_barrier()

        def group_body(g, carry):
            g0 = g * _G
            ng = jnp.clip(nk - g0, 0, _G)

            @pl.when(ng > 0)
            def _():
                pltpu.sync_copy(row_hbm.at[pl.ds(base + g0, _G)], idxr_g)
                pltpu.sync_copy(col_hbm.at[pl.ds(base + g0, _G)], idxc_g)
                for b in range(_NBUF):
                    @pl.when(b < ng)
                    def _(b=b):
                        pltpu.async_copy(
                            ypa_hbm.at[idxc_g.at[b]], sta.at[b], gsems[b])
                        pltpu.async_copy(
                            ypb_hbm.at[idxc_g.at[b]], stb.at[b], gsems[b])

                def round_body(q, carry2):
                    for u in range(_NBB):
                        kk = q * _NBB + u
                        b = u % _NBUF

                        @pl.when(kk < ng)
                        def _(b=b, u=u, kk=kk):
                            pltpu.make_async_copy(
                                ypa_hbm.at[idxc_g.at[kk]], sta.at[b],
                                gsems[b]).wait()
                            pltpu.make_async_copy(
                                ypb_hbm.at[idxc_g.at[kk]], stb.at[u],
                                gsems[b]).wait()
                            pltpu.sync_copy(
                                sta.at[b], acca.at[idxr_g.at[kk]], add=True)
                            pltpu.async_copy(
                                stb.at[u], accb.at[idxr_g.at[kk]],
                                bsems[u], add=True)

                            kn = kk + _NBUF
                            un = (u + _NBUF) % _NBB

                            @pl.when(kn < ng)
                            def _():
                                pltpu.async_copy(
                                    ypa_hbm.at[idxc_g.at[kn]], sta.at[b],
                                    gsems[b])

                                @pl.when(kk >= _NBUF)
                                def _():
                                    pltpu.make_async_copy(
                                        stb.at[un],
                                        accb.at[idxr_g.at[0]],
                                        bsems[un]).wait()

                                pltpu.async_copy(
                                    ypb_hbm.at[idxc_g.at[kn]], stb.at[un],
                                    gsems[b])

                    return carry2

                lax.fori_loop(0, -(-_G // _NBB), round_body, 0)

                for u in range(_NBB):
                    @pl.when(u < ng)
                    def _(u=u):
                        pltpu.make_async_copy(
                            stb.at[u], accb.at[idxr_g.at[0]],
                            bsems[u]).wait()

            return carry

        lax.fori_loop(0, _NG, group_body, 0)
        plsc.subcore_barrier()
        for j in range(rpt // _CH):
            rb = sid * rpt + j * _CH
            pltpu.sync_copy(acca.at[pl.ds(rb, _CH)], sta.at[0])
            pltpu.sync_copy(sta.at[0], outa_hbm.at[cid, pl.ds(rb, _CH)])
            pltpu.sync_copy(accb.at[pl.ds(rb, _CH)], stb.at[0])
            pltpu.sync_copy(stb.at[0], outb_hbm.at[cid, pl.ds(rb, _CH)])

    return k(ypa, ypb, row, col, z)


def _tc_prep_body(degp_ref, x_ref, m_ref, ya_ref, yb_ref):
    d16 = degp_ref[0] + degp_ref[1]
    deg = d16[:, 0:1]
    dis = jnp.where(deg > 0, lax.rsqrt(jnp.where(deg > 0, deg, 1.0)), 0.0)
    m = m_ref[...]
    ya_ref[...] = jnp.nan_to_num(x_ref[...]) * (m * dis)
    one = jnp.ones((_RB, 1), jnp.float32)
    pad = jnp.zeros((_RB, 13), jnp.float32)
    yb_ref[...] = jnp.concatenate([dis, dis * m, one, pad], axis=1)


def _tc_prep(degp, xu, mu):
    grid = _NPAD // _RB
    return pl.pallas_call(
        _tc_prep_body,
        grid=(grid,),
        in_specs=[
            pl.BlockSpec((_NC, _RB, 16), lambda i: (0, i, 0)),
            pl.BlockSpec((_RB, _D), lambda i: (i, 0)),
            pl.BlockSpec((_RB, 1), lambda i: (i, 0)),
        ],
        out_specs=[
            pl.BlockSpec((_RB, _D), lambda i: (i, 0)),
            pl.BlockSpec((_RB, 16), lambda i: (i, 0)),
        ],
        out_shape=[
            jax.ShapeDtypeStruct((_NPAD, _D), jnp.float32),
            jax.ShapeDtypeStruct((_NPAD, 16), jnp.float32),
        ],
    )(degp, xu, mu)


def _tc_final_body(acca_ref, accb_ref, w_ref, b_ref, o_ref):
    a = acca_ref[0] + acca_ref[1]
    sb = accb_ref[0] + accb_ref[1]
    s = sb[:, 0:1]
    mm = sb[:, 1:2]
    deg = sb[:, 2:3]
    dis = jnp.where(deg > 0, lax.rsqrt(jnp.where(deg > 0, deg, 1.0)), 0.0)
    scale = jnp.where(mm > 0, dis * s / jnp.where(mm > 0, mm, 1.0), 0.0)
    r = a * scale
    o_ref[...] = lax.dot_general(
        r, w_ref[...], (((1,), (1,)), ((), ())),
        preferred_element_type=jnp.float32) + b_ref[...]


def _tc_final(acca, accb, W, b2):
    grid = _NPAD // _RB
    return pl.pallas_call(
        _tc_final_body,
        grid=(grid,),
        in_specs=[
            pl.BlockSpec((_NC, _RB, _D), lambda i: (0, i, 0)),
            pl.BlockSpec((_NC, _RB, 16), lambda i: (0, i, 0)),
            pl.BlockSpec((_D, _D), lambda i: (0, 0)),
            pl.BlockSpec((1, _D), lambda i: (0, 0)),
        ],
        out_specs=pl.BlockSpec((_RB, _D), lambda i: (i, 0)),
        out_shape=jax.ShapeDtypeStruct((_N, _D), jnp.float32),
    )(acca, accb, W, b2)


def kernel(x, edge_index, train_mask, W, b):
    row2 = edge_index[0].reshape(_NCHUNK, _CH)
    col2 = edge_index[1].reshape(_NCHUNK, _CH)

    z16 = jnp.zeros((_CH, 16), jnp.float32)
    oh16 = z16.at[:, 0].set(1.0)
    degp = _sc_deg(row2, z16, oh16)

    ypa, ypb = _tc_prep(degp, x, train_mask)

    zd = jnp.zeros((_CH, _D), jnp.float32)
    acca, accb = _sc_spmm(ypa, ypb, row2, col2, zd)

    return _tc_final(acca, accb, W, b.reshape(1, _D))


# speedup vs baseline: 1.0898x; 1.0898x over previous
import functools

import jax
import jax.numpy as jnp
from jax import lax
from jax.experimental import pallas as pl
from jax.experimental.pallas import tpu as pltpu
from jax.experimental.pallas import tpu_sc as plsc

_N = 10000
_E = 320000
_D = 128
_NPAD = 10240
_DP = 144
_CH = 64
_NCHUNK = _E // _CH
_NC, _NS = 2, 16
_NW = _NC * _NS
_G = 40
_NG = 4
_KMAX = _G * _NG
_CPAD = _NW * _KMAX
_NBUF = 3
_RB = 512


def _mesh():
    return plsc.VectorSubcoreMesh(
        core_axis_name="c", subcore_axis_name="s",
        num_cores=_NC, num_subcores=_NS)


def _sc_deg(row, z, oh):
    @functools.partial(
        pl.kernel,
        out_type=jax.ShapeDtypeStruct((_NC, _NPAD, 16), jnp.float32),
        mesh=_mesh(),
        compiler_params=pltpu.CompilerParams(use_tc_tiling_on_sc=False),
        scratch_types=[
            pltpu.VMEM((_KMAX, _CH), jnp.int32),
            pltpu.VMEM((_CH, 16), jnp.float32),
            pltpu.VMEM_SHARED((_NPAD, 16), jnp.float32),
            pltpu.SemaphoreType.DMA,
        ],
    )
    def k(row_hbm, z_hbm, oh_hbm, out_hbm, idxrall, stage, dacc, sem):
        cid = lax.axis_index("c")
        sid = lax.axis_index("s")
        wid = cid * _NS + sid
        rpt = _NPAD // _NS
        base = wid * _KMAX
        nk = jnp.minimum(_KMAX, _NCHUNK - base)
        pltpu.sync_copy(row_hbm.at[pl.ds(base, _KMAX)], idxrall)
        pltpu.sync_copy(z_hbm, stage)
        for j in range(rpt // _CH):
            pltpu.sync_copy(stage, dacc.at[pl.ds(sid * rpt + j * _CH, _CH)])
        pltpu.sync_copy(oh_hbm, stage)
        plsc.subcore_barrier()

        def body(kk, carry):
            @pl.when(kk < nk)
            def _():
                pltpu.async_copy(stage, dacc.at[idxrall.at[kk]], sem, add=True)

            return carry

        lax.fori_loop(0, _KMAX, body, 0)

        def drain(kk, carry):
            @pl.when(kk < nk)
            def _():
                pltpu.make_async_copy(stage, dacc.at[idxrall.at[0]],
                                      sem).wait()

            return carry

        lax.fori_loop(0, _KMAX, drain, 0)
        plsc.subcore_barrier()
        for j in range(rpt // _CH):
            base = sid * rpt + j * _CH
            pltpu.sync_copy(dacc.at[pl.ds(base, _CH)], stage)
            pltpu.sync_copy(stage, out_hbm.at[cid, pl.ds(base, _CH)])

    return k(row, z, oh)


_NBB = 2 * _NBUF


def _sc_spmm(ypa, ypb, row, col, z):
    @functools.partial(
        pl.kernel,
        out_type=[
            jax.ShapeDtypeStruct((_NC, _NPAD, _D), jnp.float32),
            jax.ShapeDtypeStruct((_NC, _NPAD, 16), jnp.float32),
        ],
        mesh=_mesh(),
        compiler_params=pltpu.CompilerParams(use_tc_tiling_on_sc=False),
        scratch_types=[
            pltpu.VMEM((_G, _CH), jnp.int32),
            pltpu.VMEM((_G, _CH), jnp.int32),
            pltpu.VMEM((_NBUF, _CH, _D), jnp.float32),
            pltpu.VMEM((_NBB, _CH, 16), jnp.float32),
            pltpu.VMEM_SHARED((_NPAD, _D), jnp.float32),
            pltpu.VMEM_SHARED((_NPAD, 16), jnp.float32),
        ] + [pltpu.SemaphoreType.DMA] * (_NBUF + _NBB),
    )
    def k(ypa_hbm, ypb_hbm, row_hbm, col_hbm, z_hbm, outa_hbm, outb_hbm,
          idxr_g, idxc_g, sta, stb, acca, accb, *sems):
        gsems = sems[:_NBUF]
        bsems = sems[_NBUF:]
        cid = lax.axis_index("c")
        sid = lax.axis_index("s")
        wid = cid * _NS + sid
        rpt = _NPAD // _NS
        base = wid * _KMAX
        nk = jnp.minimum(_KMAX, _NCHUNK - base)
        pltpu.sync_copy(z_hbm, sta.at[0])
        pltpu.sync_copy(z_hbm.at[:, pl.ds(0, 16)], stb.at[0])
        for j in range(rpt // _CH):
            rb = sid * rpt + j * _CH
            pltpu.sync_copy(sta.at[0], acca.at[pl.ds(rb, _CH)])
            pltpu.sync_copy(stb.at[0], accb.at[pl.ds(rb, _CH)])
        plsc.subcore_barrier()

        def group_body(g, carry):
            g0 = g * _G
            ng = jnp.clip(nk - g0, 0, _G)

            @pl.when(ng > 0)
            def _():
                pltpu.sync_copy(row_hbm.at[pl.ds(base + g0, _G)], idxr_g)
                pltpu.sync_copy(col_hbm.at[pl.ds(base + g0, _G)], idxc_g)
                for b in range(_NBUF):
                    @pl.when(b < ng)
                    def _(b=b):
                        pltpu.async_copy(
                            ypa_hbm.at[idxc_g.at[b]], sta.at[b], gsems[b])
                        pltpu.async_copy(
                            ypb_hbm.at[idxc_g.at[b]], stb.at[b], gsems[b])

                def round_body(q, carry2):
                    for u in range(_NBB):
                        kk = q * _NBB + u
                        b = u % _NBUF

                        @pl.when(kk < ng)
                        def _(b=b, u=u, kk=kk):
                            pltpu.make_async_copy(
                                ypa_hbm.at[idxc_g.at[kk]], sta.at[b],
                                gsems[b]).wait()
                            pltpu.make_async_copy(
                                ypb_hbm.at[idxc_g.at[kk]], stb.at[u],
                                gsems[b]).wait()
                            pltpu.sync_copy(
                                sta.at[b], acca.at[idxr_g.at[kk]], add=True)
                            pltpu.async_copy(
                                stb.at[u], accb.at[idxr_g.at[kk]],
                                bsems[u], add=True)

                            kn = kk + _NBUF
                            un = (u + _NBUF) % _NBB

                            @pl.when(kn < ng)
                            def _():
                                pltpu.async_copy(
                                    ypa_hbm.at[idxc_g.at[kn]], sta.at[b],
                                    gsems[b])

                                @pl.when(kk >= _NBUF)
                                def _():
                                    pltpu.make_async_copy(
                                        stb.at[un],
                                        accb.at[idxr_g.at[0]],
                                        bsems[un]).wait()

                                pltpu.async_copy(
                                    ypb_hbm.at[idxc_g.at[kn]], stb.at[un],
                                    gsems[b])

                    return carry2

                lax.fori_loop(0, -(-_G // _NBB), round_body, 0)

                for u in range(_NBB):
                    @pl.when(u < ng)
                    def _(u=u):
                        pltpu.make_async_copy(
                            stb.at[u], accb.at[idxr_g.at[0]],
                            bsems[u]).wait()

            return carry

        lax.fori_loop(0, _NG, group_body, 0)
        plsc.subcore_barrier()
        for j in range(rpt // _CH):
            rb = sid * rpt + j * _CH
            pltpu.sync_copy(acca.at[pl.ds(rb, _CH)], sta.at[0])
            pltpu.sync_copy(sta.at[0], outa_hbm.at[cid, pl.ds(rb, _CH)])
            pltpu.sync_copy(accb.at[pl.ds(rb, _CH)], stb.at[0])
            pltpu.sync_copy(stb.at[0], outb_hbm.at[cid, pl.ds(rb, _CH)])

    return k(ypa, ypb, row, col, z)


def _tc_prep_body(degp_ref, x_ref, m_ref, ya_ref, yb_ref):
    d16 = degp_ref[0] + degp_ref[1]
    deg = d16[:, 0:1]
    dis = jnp.where(deg > 0, lax.rsqrt(jnp.where(deg > 0, deg, 1.0)), 0.0)
    m = m_ref[...]
    ya_ref[...] = jnp.nan_to_num(x_ref[...]) * (m * dis)
    one = jnp.ones((_RB, 1), jnp.float32)
    pad = jnp.zeros((_RB, 13), jnp.float32)
    yb_ref[...] = jnp.concatenate([dis, dis * m, one, pad], axis=1)


def _tc_prep(degp, xu, mu):
    grid = _NPAD // _RB
    return pl.pallas_call(
        _tc_prep_body,
        grid=(grid,),
        in_specs=[
            pl.BlockSpec((_NC, _RB, 16), lambda i: (0, i, 0)),
            pl.BlockSpec((_RB, _D), lambda i: (i, 0)),
            pl.BlockSpec((_RB, 1), lambda i: (i, 0)),
        ],
        out_specs=[
            pl.BlockSpec((_RB, _D), lambda i: (i, 0)),
            pl.BlockSpec((_RB, 16), lambda i: (i, 0)),
        ],
        out_shape=[
            jax.ShapeDtypeStruct((_NPAD, _D), jnp.float32),
            jax.ShapeDtypeStruct((_NPAD, 16), jnp.float32),
        ],
    )(degp, xu, mu)


def _tc_final_body(acca_ref, accb_ref, w_ref, b_ref, o_ref):
    a = acca_ref[0] + acca_ref[1]
    sb = accb_ref[0] + accb_ref[1]
    s = sb[:, 0:1]
    mm = sb[:, 1:2]
    deg = sb[:, 2:3]
    dis = jnp.where(deg > 0, lax.rsqrt(jnp.where(deg > 0, deg, 1.0)), 0.0)
    scale = jnp.where(mm > 0, dis * s / jnp.where(mm > 0, mm, 1.0), 0.0)
    r = a * scale
    o_ref[...] = lax.dot_general(
        r, w_ref[...], (((1,), (1,)), ((), ())),
        preferred_element_type=jnp.float32) + b_ref[...]


def _tc_final(acca, accb, W, b2):
    grid = _NPAD // _RB
    return pl.pallas_call(
        _tc_final_body,
        grid=(grid,),
        in_specs=[
            pl.BlockSpec((_NC, _RB, _D), lambda i: (0, i, 0)),
            pl.BlockSpec((_NC, _RB, 16), lambda i: (0, i, 0)),
            pl.BlockSpec((_D, _D), lambda i: (0, 0)),
            pl.BlockSpec((1, _D), lambda i: (0, 0)),
        ],
        out_specs=pl.BlockSpec((_RB, _D), lambda i: (i, 0)),
        out_shape=jax.ShapeDtypeStruct((_N, _D), jnp.float32),
    )(acca, accb, W, b2)


def kernel(x, edge_index, train_mask, W, b):
    row2 = edge_index[0].reshape(_NCHUNK, _CH)
    col2 = edge_index[1].reshape(_NCHUNK, _CH)

    z16 = jnp.zeros((_CH, 16), jnp.float32)
    oh16 = z16.at[:, 0].set(1.0)
    degp = _sc_deg(row2, z16, oh16)
    col2, degp = lax.optimization_barrier((col2, degp))

    ypa, ypb = _tc_prep(degp, x, train_mask)

    zd = jnp.zeros((_CH, _D), jnp.float32)
    acca, accb = _sc_spmm(ypa, ypb, row2, col2, zd)

    return _tc_final(acca, accb, W, b.reshape(1, _D))


# speedup vs baseline: 1.1441x; 1.0498x over previous
import functools

import jax
import jax.numpy as jnp
from jax import lax
from jax.experimental import pallas as pl
from jax.experimental.pallas import tpu as pltpu
from jax.experimental.pallas import tpu_sc as plsc

_N = 10000
_E = 320000
_D = 128
_NPAD = 10240
_DP = 144
_CH = 64
_NCHUNK = _E // _CH
_NC, _NS = 2, 16
_NW = _NC * _NS
_G = 40
_NG = 4
_KMAX = _G * _NG
_CPAD = _NW * _KMAX
_NBUF = 3
_RB = 1024


def _mesh():
    return plsc.VectorSubcoreMesh(
        core_axis_name="c", subcore_axis_name="s",
        num_cores=_NC, num_subcores=_NS)


def _sc_deg(row, z, oh):
    @functools.partial(
        pl.kernel,
        out_type=jax.ShapeDtypeStruct((_NC, _NPAD, 16), jnp.float32),
        mesh=_mesh(),
        compiler_params=pltpu.CompilerParams(use_tc_tiling_on_sc=False),
        scratch_types=[
            pltpu.VMEM((_KMAX, _CH), jnp.int32),
            pltpu.VMEM((_CH, 16), jnp.float32),
            pltpu.VMEM_SHARED((_NPAD, 16), jnp.float32),
            pltpu.SemaphoreType.DMA,
        ],
    )
    def k(row_hbm, z_hbm, oh_hbm, out_hbm, idxrall, stage, dacc, sem):
        cid = lax.axis_index("c")
        sid = lax.axis_index("s")
        wid = cid * _NS + sid
        rpt = _NPAD // _NS
        base = wid * _KMAX
        nk = jnp.minimum(_KMAX, _NCHUNK - base)
        pltpu.sync_copy(row_hbm.at[pl.ds(base, _KMAX)], idxrall)
        pltpu.sync_copy(z_hbm, stage)
        for j in range(rpt // _CH):
            pltpu.sync_copy(stage, dacc.at[pl.ds(sid * rpt + j * _CH, _CH)])
        pltpu.sync_copy(oh_hbm, stage)
        plsc.subcore_barrier()

        def body(kk, carry):
            @pl.when(kk < nk)
            def _():
                pltpu.async_copy(stage, dacc.at[idxrall.at[kk]], sem, add=True)

            return carry

        lax.fori_loop(0, _KMAX, body, 0)

        def drain(kk, carry):
            @pl.when(kk < nk)
            def _():
                pltpu.make_async_copy(stage, dacc.at[idxrall.at[0]],
                                      sem).wait()

            return carry

        lax.fori_loop(0, _KMAX, drain, 0)
        plsc.subcore_barrier()
        for j in range(rpt // _CH):
            base = sid * rpt + j * _CH
            pltpu.sync_copy(dacc.at[pl.ds(base, _CH)], stage)
            pltpu.sync_copy(stage, out_hbm.at[cid, pl.ds(base, _CH)])

    return k(row, z, oh)


_NBB = 2 * _NBUF


def _sc_spmm(ypa, ypb, row, col, z):
    @functools.partial(
        pl.kernel,
        out_type=[
            jax.ShapeDtypeStruct((_NC, _NPAD, _D), jnp.float32),
            jax.ShapeDtypeStruct((_NC, _NPAD, 16), jnp.float32),
        ],
        mesh=_mesh(),
        compiler_params=pltpu.CompilerParams(use_tc_tiling_on_sc=False),
        scratch_types=[
            pltpu.VMEM((_G, _CH), jnp.int32),
            pltpu.VMEM((_G, _CH), jnp.int32),
            pltpu.VMEM((_NBUF, _CH, _D), jnp.float32),
            pltpu.VMEM((_NBB, _CH, 16), jnp.float32),
            pltpu.VMEM_SHARED((_NPAD, _D), jnp.float32),
            pltpu.VMEM_SHARED((_NPAD, 16), jnp.float32),
        ] + [pltpu.SemaphoreType.DMA] * (_NBUF + _NBB),
    )
    def k(ypa_hbm, ypb_hbm, row_hbm, col_hbm, z_hbm, outa_hbm, outb_hbm,
          idxr_g, idxc_g, sta, stb, acca, accb, *sems):
        gsems = sems[:_NBUF]
        bsems = sems[_NBUF:]
        cid = lax.axis_index("c")
        sid = lax.axis_index("s")
        wid = cid * _NS + sid
        rpt = _NPAD // _NS
        base = wid * _KMAX
        nk = jnp.minimum(_KMAX, _NCHUNK - base)
        pltpu.sync_copy(z_hbm, sta.at[0])
        pltpu.sync_copy(z_hbm.at[:, pl.ds(0, 16)], stb.at[0])
        for j in range(rpt // _CH):
            rb = sid * rpt + j * _CH
            pltpu.sync_copy(sta.at[0], acca.at[pl.ds(rb, _CH)])
            pltpu.sync_copy(stb.at[0], accb.at[pl.ds(rb, _CH)])
        plsc.subcore_barrier()

        def group_body(g, carry):
            g0 = g * _G
            ng = jnp.clip(nk - g0, 0, _G)

            @pl.when(ng > 0)
            def _():
                pltpu.sync_copy(row_hbm.at[pl.ds(base + g0, _G)], idxr_g)
                pltpu.sync_copy(col_hbm.at[pl.ds(base + g0, _G)], idxc_g)
                for b in range(_NBUF):
                    @pl.when(b < ng)
                    def _(b=b):
                        pltpu.async_copy(
                            ypa_hbm.at[idxc_g.at[b]], sta.at[b], gsems[b])
                        pltpu.async_copy(
                            ypb_hbm.at[idxc_g.at[b]], stb.at[b], gsems[b])

                def round_body(q, carry2):
                    for u in range(_NBB):
                        kk = q * _NBB + u
                        b = u % _NBUF

                        @pl.when(kk < ng)
                        def _(b=b, u=u, kk=kk):
                            pltpu.make_async_copy(
                                ypa_hbm.at[idxc_g.at[kk]], sta.at[b],
                                gsems[b]).wait()
                            pltpu.make_async_copy(
                                ypb_hbm.at[idxc_g.at[kk]], stb.at[u],
                                gsems[b]).wait()
                            pltpu.sync_copy(
                                sta.at[b], acca.at[idxr_g.at[kk]], add=True)
                            pltpu.async_copy(
                                stb.at[u], accb.at[idxr_g.at[kk]],
                                bsems[u], add=True)

                            kn = kk + _NBUF
                            un = (u + _NBUF) % _NBB

                            @pl.when(kn < ng)
                            def _():
                                pltpu.async_copy(
                                    ypa_hbm.at[idxc_g.at[kn]], sta.at[b],
                                    gsems[b])

                                @pl.when(kk >= _NBUF)
                                def _():
                                    pltpu.make_async_copy(
                                        stb.at[un],
                                        accb.at[idxr_g.at[0]],
                                        bsems[un]).wait()

                                pltpu.async_copy(
                                    ypb_hbm.at[idxc_g.at[kn]], stb.at[un],
                                    gsems[b])

                    return carry2

                lax.fori_loop(0, -(-_G // _NBB), round_body, 0)

                for u in range(_NBB):
                    @pl.when(u < ng)
                    def _(u=u):
                        pltpu.make_async_copy(
                            stb.at[u], accb.at[idxr_g.at[0]],
                            bsems[u]).wait()

            return carry

        lax.fori_loop(0, _NG, group_body, 0)
        plsc.subcore_barrier()
        for j in range(rpt // _CH):
            rb = sid * rpt + j * _CH
            pltpu.sync_copy(acca.at[pl.ds(rb, _CH)], sta.at[0])
            pltpu.sync_copy(sta.at[0], outa_hbm.at[cid, pl.ds(rb, _CH)])
            pltpu.sync_copy(accb.at[pl.ds(rb, _CH)], stb.at[0])
            pltpu.sync_copy(stb.at[0], outb_hbm.at[cid, pl.ds(rb, _CH)])

    return k(ypa, ypb, row, col, z)


def _tc_prep_body(degp_ref, x_ref, m_ref, ya_ref, yb_ref):
    d16 = degp_ref[0] + degp_ref[1]
    deg = d16[:, 0:1]
    dis = jnp.where(deg > 0, lax.rsqrt(jnp.where(deg > 0, deg, 1.0)), 0.0)
    m = m_ref[...]
    ya_ref[...] = jnp.nan_to_num(x_ref[...]) * (m * dis)
    one = jnp.ones((_RB, 1), jnp.float32)
    pad = jnp.zeros((_RB, 13), jnp.float32)
    yb_ref[...] = jnp.concatenate([dis, dis * m, one, pad], axis=1)


def _tc_prep(degp, xu, mu):
    grid = _NPAD // _RB
    return pl.pallas_call(
        _tc_prep_body,
        grid=(grid,),
        in_specs=[
            pl.BlockSpec((_NC, _RB, 16), lambda i: (0, i, 0)),
            pl.BlockSpec((_RB, _D), lambda i: (i, 0)),
            pl.BlockSpec((_RB, 1), lambda i: (i, 0)),
        ],
        out_specs=[
            pl.BlockSpec((_RB, _D), lambda i: (i, 0)),
            pl.BlockSpec((_RB, 16), lambda i: (i, 0)),
        ],
        out_shape=[
            jax.ShapeDtypeStruct((_NPAD, _D), jnp.float32),
            jax.ShapeDtypeStruct((_NPAD, 16), jnp.float32),
        ],
    )(degp, xu, mu)


def _tc_final_body(acca_ref, accb_ref, w_ref, b_ref, o_ref):
    a = acca_ref[0] + acca_ref[1]
    sb = accb_ref[0] + accb_ref[1]
    s = sb[:, 0:1]
    mm = sb[:, 1:2]
    deg = sb[:, 2:3]
    dis = jnp.where(deg > 0, lax.rsqrt(jnp.where(deg > 0, deg, 1.0)), 0.0)
    scale = jnp.where(mm > 0, dis * s / jnp.where(mm > 0, mm, 1.0), 0.0)
    r = a * scale
    o_ref[...] = lax.dot_general(
        r, w_ref[...], (((1,), (1,)), ((), ())),
        preferred_element_type=jnp.float32) + b_ref[...]


def _tc_final(acca, accb, W, b2):
    grid = _NPAD // _RB
    return pl.pallas_call(
        _tc_final_body,
        grid=(grid,),
        in_specs=[
            pl.BlockSpec((_NC, _RB, _D), lambda i: (0, i, 0)),
            pl.BlockSpec((_NC, _RB, 16), lambda i: (0, i, 0)),
            pl.BlockSpec((_D, _D), lambda i: (0, 0)),
            pl.BlockSpec((1, _D), lambda i: (0, 0)),
        ],
        out_specs=pl.BlockSpec((_RB, _D), lambda i: (i, 0)),
        out_shape=jax.ShapeDtypeStruct((_N, _D), jnp.float32),
    )(acca, accb, W, b2)


def kernel(x, edge_index, train_mask, W, b):
    row2 = edge_index[0].reshape(_NCHUNK, _CH)
    col2 = edge_index[1].reshape(_NCHUNK, _CH)

    z16 = jnp.zeros((_CH, 16), jnp.float32)
    oh16 = z16.at[:, 0].set(1.0)
    degp = _sc_deg(row2, z16, oh16)
    col2, degp = lax.optimization_barrier((col2, degp))

    ypa, ypb = _tc_prep(degp, x, train_mask)

    zd = jnp.zeros((_CH, _D), jnp.float32)
    acca, accb = _sc_spmm(ypa, ypb, row2, col2, zd)

    return _tc_final(acca, accb, W, b.reshape(1, _D))


# speedup vs baseline: 1.1634x; 1.0169x over previous
import functools

import jax
import jax.numpy as jnp
from jax import lax
from jax.experimental import pallas as pl
from jax.experimental.pallas import tpu as pltpu
from jax.experimental.pallas import tpu_sc as plsc

_N = 10000
_E = 320000
_D = 128
_NPAD = 10240
_DP = 144
_CH = 64
_NCHUNK = _E // _CH
_NC, _NS = 2, 16
_NW = _NC * _NS
_G = 40
_NG = 4
_KMAX = _G * _NG
_CPAD = _NW * _KMAX
_NBUF = 3
_RB = 2048


def _mesh():
    return plsc.VectorSubcoreMesh(
        core_axis_name="c", subcore_axis_name="s",
        num_cores=_NC, num_subcores=_NS)


def _sc_deg(row, z, oh):
    @functools.partial(
        pl.kernel,
        out_type=jax.ShapeDtypeStruct((_NC, _NPAD, 16), jnp.float32),
        mesh=_mesh(),
        compiler_params=pltpu.CompilerParams(use_tc_tiling_on_sc=False),
        scratch_types=[
            pltpu.VMEM((_KMAX, _CH), jnp.int32),
            pltpu.VMEM((_CH, 16), jnp.float32),
            pltpu.VMEM_SHARED((_NPAD, 16), jnp.float32),
            pltpu.SemaphoreType.DMA,
        ],
    )
    def k(row_hbm, z_hbm, oh_hbm, out_hbm, idxrall, stage, dacc, sem):
        cid = lax.axis_index("c")
        sid = lax.axis_index("s")
        wid = cid * _NS + sid
        rpt = _NPAD // _NS
        base = wid * _KMAX
        nk = jnp.minimum(_KMAX, _NCHUNK - base)
        pltpu.sync_copy(row_hbm.at[pl.ds(base, _KMAX)], idxrall)
        pltpu.sync_copy(z_hbm, stage)
        for j in range(rpt // _CH):
            pltpu.sync_copy(stage, dacc.at[pl.ds(sid * rpt + j * _CH, _CH)])
        pltpu.sync_copy(oh_hbm, stage)
        plsc.subcore_barrier()

        def body(kk, carry):
            @pl.when(kk < nk)
            def _():
                pltpu.async_copy(stage, dacc.at[idxrall.at[kk]], sem, add=True)

            return carry

        lax.fori_loop(0, _KMAX, body, 0)

        def drain(kk, carry):
            @pl.when(kk < nk)
            def _():
                pltpu.make_async_copy(stage, dacc.at[idxrall.at[0]],
                                      sem).wait()

            return carry

        lax.fori_loop(0, _KMAX, drain, 0)
        plsc.subcore_barrier()
        for j in range(rpt // _CH):
            base = sid * rpt + j * _CH
            pltpu.sync_copy(dacc.at[pl.ds(base, _CH)], stage)
            pltpu.sync_copy(stage, out_hbm.at[cid, pl.ds(base, _CH)])

    return k(row, z, oh)


_NBB = 2 * _NBUF


def _sc_spmm(ypa, ypb, row, col, z):
    @functools.partial(
        pl.kernel,
        out_type=[
            jax.ShapeDtypeStruct((_NC, _NPAD, _D), jnp.float32),
            jax.ShapeDtypeStruct((_NC, _NPAD, 16), jnp.float32),
        ],
        mesh=_mesh(),
        compiler_params=pltpu.CompilerParams(use_tc_tiling_on_sc=False),
        scratch_types=[
            pltpu.VMEM((_G, _CH), jnp.int32),
            pltpu.VMEM((_G, _CH), jnp.int32),
            pltpu.VMEM((_NBUF, _CH, _D), jnp.float32),
            pltpu.VMEM((_NBB, _CH, 16), jnp.float32),
            pltpu.VMEM_SHARED((_NPAD, _D), jnp.float32),
            pltpu.VMEM_SHARED((_NPAD, 16), jnp.float32),
        ] + [pltpu.SemaphoreType.DMA] * (_NBUF + _NBB),
    )
    def k(ypa_hbm, ypb_hbm, row_hbm, col_hbm, z_hbm, outa_hbm, outb_hbm,
          idxr_g, idxc_g, sta, stb, acca, accb, *sems):
        gsems = sems[:_NBUF]
        bsems = sems[_NBUF:]
        cid = lax.axis_index("c")
        sid = lax.axis_index("s")
        wid = cid * _NS + sid
        rpt = _NPAD // _NS
        base = wid * _KMAX
        nk = jnp.minimum(_KMAX, _NCHUNK - base)
        pltpu.sync_copy(z_hbm, sta.at[0])
        pltpu.sync_copy(z_hbm.at[:, pl.ds(0, 16)], stb.at[0])
        for j in range(rpt // _CH):
            rb = sid * rpt + j * _CH
            pltpu.sync_copy(sta.at[0], acca.at[pl.ds(rb, _CH)])
            pltpu.sync_copy(stb.at[0], accb.at[pl.ds(rb, _CH)])
        plsc.subcore_barrier()

        def group_body(g, carry):
            g0 = g * _G
            ng = jnp.clip(nk - g0, 0, _G)

            @pl.when(ng > 0)
            def _():
                pltpu.sync_copy(row_hbm.at[pl.ds(base + g0, _G)], idxr_g)
                pltpu.sync_copy(col_hbm.at[pl.ds(base + g0, _G)], idxc_g)
                for b in range(_NBUF):
                    @pl.when(b < ng)
                    def _(b=b):
                        pltpu.async_copy(
                            ypa_hbm.at[idxc_g.at[b]], sta.at[b], gsems[b])
                        pltpu.async_copy(
                            ypb_hbm.at[idxc_g.at[b]], stb.at[b], gsems[b])

                def round_body(q, carry2):
                    for u in range(_NBB):
                        kk = q * _NBB + u
                        b = u % _NBUF

                        @pl.when(kk < ng)
                        def _(b=b, u=u, kk=kk):
                            pltpu.make_async_copy(
                                ypa_hbm.at[idxc_g.at[kk]], sta.at[b],
                                gsems[b]).wait()
                            pltpu.make_async_copy(
                                ypb_hbm.at[idxc_g.at[kk]], stb.at[u],
                                gsems[b]).wait()
                            pltpu.sync_copy(
                                sta.at[b], acca.at[idxr_g.at[kk]], add=True)
                            pltpu.async_copy(
                                stb.at[u], accb.at[idxr_g.at[kk]],
                                bsems[u], add=True)

                            kn = kk + _NBUF
                            un = (u + _NBUF) % _NBB

                            @pl.when(kn < ng)
                            def _():
                                pltpu.async_copy(
                                    ypa_hbm.at[idxc_g.at[kn]], sta.at[b],
                                    gsems[b])

                                @pl.when(kk >= _NBUF)
                                def _():
                                    pltpu.make_async_copy(
                                        stb.at[un],
                                        accb.at[idxr_g.at[0]],
                                        bsems[un]).wait()

                                pltpu.async_copy(
                                    ypb_hbm.at[idxc_g.at[kn]], stb.at[un],
                                    gsems[b])

                    return carry2

                lax.fori_loop(0, -(-_G // _NBB), round_body, 0)

                for u in range(_NBB):
                    @pl.when(u < ng)
                    def _(u=u):
                        pltpu.make_async_copy(
                            stb.at[u], accb.at[idxr_g.at[0]],
                            bsems[u]).wait()

            return carry

        lax.fori_loop(0, _NG, group_body, 0)
        plsc.subcore_barrier()
        for j in range(rpt // _CH):
            rb = sid * rpt + j * _CH
            pltpu.sync_copy(acca.at[pl.ds(rb, _CH)], sta.at[0])
            pltpu.sync_copy(sta.at[0], outa_hbm.at[cid, pl.ds(rb, _CH)])
            pltpu.sync_copy(accb.at[pl.ds(rb, _CH)], stb.at[0])
            pltpu.sync_copy(stb.at[0], outb_hbm.at[cid, pl.ds(rb, _CH)])

    return k(ypa, ypb, row, col, z)


def _tc_prep_body(degp_ref, x_ref, m_ref, ya_ref, yb_ref):
    d16 = degp_ref[0] + degp_ref[1]
    deg = d16[:, 0:1]
    dis = jnp.where(deg > 0, lax.rsqrt(jnp.where(deg > 0, deg, 1.0)), 0.0)
    m = m_ref[...]
    ya_ref[...] = jnp.nan_to_num(x_ref[...]) * (m * dis)
    one = jnp.ones((_RB, 1), jnp.float32)
    pad = jnp.zeros((_RB, 13), jnp.float32)
    yb_ref[...] = jnp.concatenate([dis, dis * m, one, pad], axis=1)


def _tc_prep(degp, xu, mu):
    grid = _NPAD // _RB
    return pl.pallas_call(
        _tc_prep_body,
        grid=(grid,),
        in_specs=[
            pl.BlockSpec((_NC, _RB, 16), lambda i: (0, i, 0)),
            pl.BlockSpec((_RB, _D), lambda i: (i, 0)),
            pl.BlockSpec((_RB, 1), lambda i: (i, 0)),
        ],
        out_specs=[
            pl.BlockSpec((_RB, _D), lambda i: (i, 0)),
            pl.BlockSpec((_RB, 16), lambda i: (i, 0)),
        ],
        out_shape=[
            jax.ShapeDtypeStruct((_NPAD, _D), jnp.float32),
            jax.ShapeDtypeStruct((_NPAD, 16), jnp.float32),
        ],
    )(degp, xu, mu)


def _tc_final_body(acca_ref, accb_ref, w_ref, b_ref, o_ref):
    a = acca_ref[0] + acca_ref[1]
    sb = accb_ref[0] + accb_ref[1]
    s = sb[:, 0:1]
    mm = sb[:, 1:2]
    deg = sb[:, 2:3]
    dis = jnp.where(deg > 0, lax.rsqrt(jnp.where(deg > 0, deg, 1.0)), 0.0)
    scale = jnp.where(mm > 0, dis * s / jnp.where(mm > 0, mm, 1.0), 0.0)
    r = a * scale
    o_ref[...] = lax.dot_general(
        r, w_ref[...], (((1,), (1,)), ((), ())),
        preferred_element_type=jnp.float32) + b_ref[...]


def _tc_final(acca, accb, W, b2):
    grid = _NPAD // _RB
    return pl.pallas_call(
        _tc_final_body,
        grid=(grid,),
        in_specs=[
            pl.BlockSpec((_NC, _RB, _D), lambda i: (0, i, 0)),
            pl.BlockSpec((_NC, _RB, 16), lambda i: (0, i, 0)),
            pl.BlockSpec((_D, _D), lambda i: (0, 0)),
            pl.BlockSpec((1, _D), lambda i: (0, 0)),
        ],
        out_specs=pl.BlockSpec((_RB, _D), lambda i: (i, 0)),
        out_shape=jax.ShapeDtypeStruct((_N, _D), jnp.float32),
    )(acca, accb, W, b2)


def kernel(x, edge_index, train_mask, W, b):
    row2 = edge_index[0].reshape(_NCHUNK, _CH)
    col2 = edge_index[1].reshape(_NCHUNK, _CH)

    z16 = jnp.zeros((_CH, 16), jnp.float32)
    oh16 = z16.at[:, 0].set(1.0)
    degp = _sc_deg(row2, z16, oh16)
    col2, degp = lax.optimization_barrier((col2, degp))

    ypa, ypb = _tc_prep(degp, x, train_mask)

    zd = jnp.zeros((_CH, _D), jnp.float32)
    acca, accb = _sc_spmm(ypa, ypb, row2, col2, zd)

    return _tc_final(acca, accb, W, b.reshape(1, _D))
